# Initial kernel scaffold; baseline (speedup 1.0000x reference)
import jax
import jax.numpy as jnp
from jax.experimental import pallas as pl


def kernel(x, edge_index, embed, W1, b1, W2, b2):
    raise NotImplementedError("write your pallas kernel here")


# speedup vs baseline: 16.0064x; 16.0064x over previous
import functools

import jax
import jax.numpy as jnp
from jax import lax
from jax.experimental import pallas as pl
from jax.experimental.pallas import tpu as pltpu
from jax.experimental.pallas import tpu_sc as plsc

NC = 2
NS = 16
NW = NC * NS
CHUNK = 128
DEG_LANES = 16
NPAD = 10240


def _vector_mesh():
    return plsc.VectorSubcoreMesh(core_axis_name="c", subcore_axis_name="s")


def _sc_degree(dst, zeros_deg):
    e = dst.shape[0]
    n = zeros_deg.shape[0]
    n_chunks = e // CHUNK
    trips = (n_chunks + NW - 1) // NW
    rpt = n // NS

    @functools.partial(
        pl.kernel,
        out_type=jax.ShapeDtypeStruct((NC, n, DEG_LANES), jnp.float32),
        mesh=_vector_mesh(),
        scratch_types=[
            pltpu.VMEM((2, CHUNK), jnp.int32),
            pltpu.VMEM((CHUNK, DEG_LANES), jnp.float32),
            pltpu.VMEM_SHARED((n, DEG_LANES), jnp.float32),
        ],
    )
    def k(dst_hbm, zeros_hbm, out_hbm, idx_v, ones_v, acc_sh):
        core = lax.axis_index("c")
        sid = lax.axis_index("s")
        wid = sid * NC + core
        r0 = sid * rpt

        @pl.loop(0, CHUNK)
        def _(r):
            ones_v[r, :] = jnp.ones((DEG_LANES,), jnp.float32)

        pltpu.sync_copy(zeros_hbm.at[pl.ds(r0, rpt)], acc_sh.at[pl.ds(r0, rpt)])
        plsc.subcore_barrier()

        @pl.loop(0, trips)
        def _(t):
            cid = t * NW + wid

            @pl.when(cid < n_chunks)
            def _():
                pltpu.sync_copy(dst_hbm.at[pl.ds(cid * CHUNK, CHUNK)], idx_v.at[0])
                pltpu.sync_copy(ones_v, acc_sh.at[idx_v.at[0]], add=True)

        plsc.subcore_barrier()
        pltpu.sync_copy(acc_sh.at[pl.ds(r0, rpt)],
                        out_hbm.at[core, pl.ds(r0, rpt)])

    return k(dst, zeros_deg)


def _sc_aggregate(g, src, dst, zeros_nd):
    d = g.shape[1]
    n = zeros_nd.shape[0]
    e = src.shape[0]
    n_chunks = e // CHUNK
    trips = (n_chunks + NW - 1) // NW
    rpt = n // NS

    @functools.partial(
        pl.kernel,
        out_type=jax.ShapeDtypeStruct((NC, n, d), jnp.float32),
        mesh=_vector_mesh(),
        scratch_types=[
            pltpu.VMEM((2, CHUNK), jnp.int32),
            pltpu.VMEM((2, CHUNK), jnp.int32),
            pltpu.VMEM((2, CHUNK, d), jnp.float32),
            pltpu.VMEM_SHARED((n, d), jnp.float32),
            pltpu.SemaphoreType.DMA,
        ],
    )
    def k(g_hbm, src_hbm, dst_hbm, zeros_hbm, out_hbm,
          srcv, dstv, rows, acc_sh, gsem):
        core = lax.axis_index("c")
        sid = lax.axis_index("s")
        wid = sid * NC + core
        r0 = sid * rpt

        pltpu.sync_copy(zeros_hbm.at[pl.ds(r0, rpt)], acc_sh.at[pl.ds(r0, rpt)])
        plsc.subcore_barrier()

        @pl.loop(0, trips)
        def _(t):
            cid = t * NW + wid

            @pl.when(cid < n_chunks)
            def _():
                base = cid * CHUNK
                pltpu.sync_copy(src_hbm.at[pl.ds(base, CHUNK)], srcv.at[0])
                pltpu.sync_copy(dst_hbm.at[pl.ds(base, CHUNK)], dstv.at[0])
                pltpu.async_copy(g_hbm.at[srcv.at[0]], rows.at[0], gsem).wait()
                pltpu.sync_copy(rows.at[0], acc_sh.at[dstv.at[0]], add=True)

        plsc.subcore_barrier()
        pltpu.sync_copy(acc_sh.at[pl.ds(r0, rpt)],
                        out_hbm.at[core, pl.ds(r0, rpt)])

    return k(g, src, dst, zeros_nd)


_BM = 2000


def _tc_matmul(x, w):
    n, d = x.shape

    def body(x_ref, w_ref, o_ref):
        o_ref[...] = lax.dot_general(
            x_ref[...], w_ref[...], (((1,), (1,)), ((), ())),
            preferred_element_type=jnp.float32)

    return pl.pallas_call(
        body,
        grid=(n // _BM,),
        in_specs=[pl.BlockSpec((_BM, d), lambda i: (i, 0)),
                  pl.BlockSpec((d, d), lambda i: (0, 0))],
        out_specs=pl.BlockSpec((_BM, d), lambda i: (i, 0)),
        out_shape=jax.ShapeDtypeStruct((n, d), jnp.float32),
    )(x, w)


def _dinv_from(dg_ref):
    deg = dg_ref[0, :, 0] + dg_ref[1, :, 0] + 1.0
    return lax.rsqrt(deg)


def _tc_scale(h, degp):
    n, d = h.shape

    def body(h_ref, dg_ref, o_ref):
        dinv = _dinv_from(dg_ref)
        o_ref[...] = h_ref[...] * dinv[:, None]

    return pl.pallas_call(
        body,
        grid=(n // _BM,),
        in_specs=[pl.BlockSpec((_BM, d), lambda i: (i, 0)),
                  pl.BlockSpec((NC, _BM, DEG_LANES), lambda i: (0, i, 0))],
        out_specs=pl.BlockSpec((_BM, d), lambda i: (i, 0)),
        out_shape=jax.ShapeDtypeStruct((n, d), jnp.float32),
    )(h, degp)


def _tc_mid(g1, accp, degp, w2, b1):
    n, d = g1.shape

    def body(g_ref, a_ref, dg_ref, w_ref, b_ref, o_ref):
        dinv = _dinv_from(dg_ref)
        s = (a_ref[0] + a_ref[1] + g_ref[...]) * dinv[:, None] + b_ref[...]
        h1 = jnp.maximum(s, 0.0)
        o_ref[...] = lax.dot_general(
            h1, w_ref[...], (((1,), (1,)), ((), ())),
            preferred_element_type=jnp.float32) * dinv[:, None]

    return pl.pallas_call(
        body,
        grid=(n // _BM,),
        in_specs=[pl.BlockSpec((_BM, d), lambda i: (i, 0)),
                  pl.BlockSpec((NC, _BM, d), lambda i: (0, i, 0)),
                  pl.BlockSpec((NC, _BM, DEG_LANES), lambda i: (0, i, 0)),
                  pl.BlockSpec((d, d), lambda i: (0, 0)),
                  pl.BlockSpec((1, d), lambda i: (0, 0))],
        out_specs=pl.BlockSpec((_BM, d), lambda i: (i, 0)),
        out_shape=jax.ShapeDtypeStruct((n, d), jnp.float32),
    )(g1, accp, degp, w2, b1)


def _tc_final(g2, accp, degp, b2):
    n, d = g2.shape

    def body(g_ref, a_ref, dg_ref, b_ref, o_ref):
        dinv = _dinv_from(dg_ref)
        s = (a_ref[0] + a_ref[1] + g_ref[...]) * dinv[:, None] + b_ref[...]
        o_ref[...] = jax.nn.sigmoid(s)

    return pl.pallas_call(
        body,
        grid=(n // _BM,),
        in_specs=[pl.BlockSpec((_BM, d), lambda i: (i, 0)),
                  pl.BlockSpec((NC, _BM, d), lambda i: (0, i, 0)),
                  pl.BlockSpec((NC, _BM, DEG_LANES), lambda i: (0, i, 0)),
                  pl.BlockSpec((1, d), lambda i: (0, 0))],
        out_specs=pl.BlockSpec((_BM, d), lambda i: (i, 0)),
        out_shape=jax.ShapeDtypeStruct((n, d), jnp.float32),
    )(g2, accp, degp, b2)


def kernel(x, edge_index, embed, W1, b1, W2, b2):
    n, d = embed.shape
    ei = edge_index.astype(jnp.int32)
    src = ei[0]
    dst = ei[1]
    zeros_nd = jnp.zeros((NPAD, d), jnp.float32)
    zeros_deg = jnp.zeros((NPAD, DEG_LANES), jnp.float32)

    degp = _sc_degree(dst, zeros_deg)
    h_lin = _tc_matmul(embed, W1)
    g1 = _tc_scale(h_lin, degp)
    acc1 = _sc_aggregate(g1, src, dst, zeros_nd)
    g2 = _tc_mid(g1, acc1, degp, W2, b1.reshape(1, d))
    acc2 = _sc_aggregate(g2, src, dst, zeros_nd)
    return _tc_final(g2, acc2, degp, b2.reshape(1, d))
